# baseline (device time: 16519 ns/iter reference)
import jax
import jax.numpy as jnp
from jax import lax
from jax.experimental import pallas as pl
from jax.experimental.pallas import tpu as pltpu

N_DEV = 8


def kernel(x, w_mat):
    k_dim, m_per = x.shape
    n_dim = w_mat.shape[1]

    def body(x_ref, w_ref, out_ref, gather_ref, send_sems, recv_sems):
        my_pos = lax.axis_index("i")

        gather_ref[my_pos] = x_ref[pl.ds(my_pos * m_per, m_per), :]

        for j in range(N_DEV):
            @pl.when(my_pos != j)
            def _():
                rdma = pltpu.make_async_remote_copy(
                    src_ref=x_ref.at[pl.ds(j * m_per, m_per), :],
                    dst_ref=gather_ref.at[my_pos],
                    send_sem=send_sems.at[j],
                    recv_sem=recv_sems.at[my_pos],
                    device_id=(j,),
                    device_id_type=pl.DeviceIdType.MESH,
                )
                rdma.start()

        for j in range(N_DEV):
            @pl.when(my_pos != j)
            def _():
                recv = pltpu.make_async_remote_copy(
                    src_ref=x_ref.at[pl.ds(j * m_per, m_per), :],
                    dst_ref=gather_ref.at[j],
                    send_sem=send_sems.at[j],
                    recv_sem=recv_sems.at[j],
                    device_id=(j,),
                    device_id_type=pl.DeviceIdType.MESH,
                )
                recv.wait_recv()

        x_rows = jnp.concatenate(
            [gather_ref[j] for j in range(N_DEV)], axis=1
        )
        out_ref[:, :] = jnp.dot(
            x_rows, w_ref[:, :], preferred_element_type=jnp.float32
        )

        for j in range(N_DEV):
            @pl.when(my_pos != j)
            def _():
                send = pltpu.make_async_remote_copy(
                    src_ref=x_ref.at[pl.ds(j * m_per, m_per), :],
                    dst_ref=gather_ref.at[j],
                    send_sem=send_sems.at[j],
                    recv_sem=recv_sems.at[j],
                    device_id=(j,),
                    device_id_type=pl.DeviceIdType.MESH,
                )
                send.wait_send()

    return pl.pallas_call(
        body,
        out_shape=jax.ShapeDtypeStruct((m_per, n_dim), jnp.float32),
        in_specs=[
            pl.BlockSpec(memory_space=pltpu.VMEM),
            pl.BlockSpec(memory_space=pltpu.VMEM),
        ],
        out_specs=pl.BlockSpec(memory_space=pltpu.VMEM),
        scratch_shapes=[
            pltpu.VMEM((N_DEV, m_per, m_per), jnp.float32),
            pltpu.SemaphoreType.DMA((N_DEV,)),
            pltpu.SemaphoreType.DMA((N_DEV,)),
        ],
    )(x, w_mat)


# device time: 16383 ns/iter; 1.0083x vs baseline; 1.0083x over previous
import jax
import jax.numpy as jnp
from jax import lax
from jax.experimental import pallas as pl
from jax.experimental.pallas import tpu as pltpu

N_DEV = 8


def kernel(x, w_mat):
    k_dim, m_per = x.shape
    n_dim = w_mat.shape[1]

    def body(x_ref, w_ref, out_ref, gather_ref, send_sems, recv_sems):
        my_pos = lax.axis_index("i")

        gather_ref[my_pos] = x_ref[pl.ds(my_pos * m_per, m_per), :]

        for j in range(N_DEV):
            @pl.when(my_pos != j)
            def _():
                rdma = pltpu.make_async_remote_copy(
                    src_ref=x_ref.at[pl.ds(j * m_per, m_per), :],
                    dst_ref=gather_ref.at[my_pos],
                    send_sem=send_sems.at[j],
                    recv_sem=recv_sems.at[my_pos],
                    device_id=(j,),
                    device_id_type=pl.DeviceIdType.MESH,
                )
                rdma.start()

        with jax.named_scope("local_gemm"):
            out_ref[:, :] = jnp.dot(
                x_ref[pl.ds(my_pos * m_per, m_per), :],
                w_ref[pl.ds(my_pos * m_per, m_per), :],
                preferred_element_type=jnp.float32,
            )

        for j in range(N_DEV):
            @pl.when(my_pos != j)
            def _():
                with jax.named_scope(f"wait_recv_{j}"):
                    recv = pltpu.make_async_remote_copy(
                        src_ref=x_ref.at[pl.ds(j * m_per, m_per), :],
                        dst_ref=gather_ref.at[j],
                        send_sem=send_sems.at[j],
                        recv_sem=recv_sems.at[j],
                        device_id=(j,),
                        device_id_type=pl.DeviceIdType.MESH,
                    )
                    recv.wait_recv()
                with jax.named_scope(f"acc_gemm_{j}"):
                    out_ref[:, :] += jnp.dot(
                        gather_ref[j],
                        w_ref[j * m_per:(j + 1) * m_per, :],
                        preferred_element_type=jnp.float32,
                    )

        for j in range(N_DEV):
            @pl.when(my_pos != j)
            def _():
                send = pltpu.make_async_remote_copy(
                    src_ref=x_ref.at[pl.ds(j * m_per, m_per), :],
                    dst_ref=gather_ref.at[j],
                    send_sem=send_sems.at[j],
                    recv_sem=recv_sems.at[j],
                    device_id=(j,),
                    device_id_type=pl.DeviceIdType.MESH,
                )
                send.wait_send()

    return pl.pallas_call(
        body,
        out_shape=jax.ShapeDtypeStruct((m_per, n_dim), jnp.float32),
        in_specs=[
            pl.BlockSpec(memory_space=pltpu.VMEM),
            pl.BlockSpec(memory_space=pltpu.VMEM),
        ],
        out_specs=pl.BlockSpec(memory_space=pltpu.VMEM),
        scratch_shapes=[
            pltpu.VMEM((N_DEV, m_per, m_per), jnp.float32),
            pltpu.SemaphoreType.DMA((N_DEV,)),
            pltpu.SemaphoreType.DMA((N_DEV,)),
        ],
    )(x, w_mat)


# device time: 13507 ns/iter; 1.2230x vs baseline; 1.2129x over previous
import jax
import jax.numpy as jnp
from jax import lax
from jax.experimental import pallas as pl
from jax.experimental.pallas import tpu as pltpu

N_DEV = 8


def kernel(x, w_mat):
    k_dim, m_per = x.shape
    n_dim = w_mat.shape[1]

    def body(x_ref, w_ref, out_ref, gather_ref, send_sems, recv_sems,
             ready_sems):
        my_pos = lax.axis_index("i")
        barrier_sem = pltpu.get_barrier_semaphore()

        for j in range(N_DEV):
            @pl.when(my_pos != j)
            def _():
                pl.semaphore_signal(
                    barrier_sem, inc=1, device_id=(j,),
                    device_id_type=pl.DeviceIdType.MESH,
                )
                pl.semaphore_signal(
                    ready_sems.at[my_pos], inc=1, device_id=(j,),
                    device_id_type=pl.DeviceIdType.MESH,
                )

        for o in range(1, N_DEV):
            dst = lax.rem(my_pos + o, N_DEV)
            pl.semaphore_wait(ready_sems.at[dst], 1)
            rdma = pltpu.make_async_remote_copy(
                src_ref=x_ref.at[pl.ds(dst * m_per, m_per), :],
                dst_ref=gather_ref.at[my_pos],
                send_sem=send_sems.at[dst],
                recv_sem=recv_sems.at[my_pos],
                device_id=(dst,),
                device_id_type=pl.DeviceIdType.MESH,
            )
            rdma.start()

        gather_ref[my_pos] = x_ref[pl.ds(my_pos * m_per, m_per), :]
        acc = jnp.dot(
            gather_ref[my_pos],
            w_ref[pl.ds(my_pos * m_per, m_per), :],
            preferred_element_type=jnp.float32,
        )

        for o in range(1, N_DEV):
            src = lax.rem(my_pos - o + N_DEV, N_DEV)
            recv = pltpu.make_async_remote_copy(
                src_ref=x_ref.at[pl.ds(src * m_per, m_per), :],
                dst_ref=gather_ref.at[src],
                send_sem=send_sems.at[src],
                recv_sem=recv_sems.at[src],
                device_id=(src,),
                device_id_type=pl.DeviceIdType.MESH,
            )
            recv.wait_recv()
            acc = acc + jnp.dot(
                gather_ref[src],
                w_ref[pl.ds(src * m_per, m_per), :],
                preferred_element_type=jnp.float32,
            )
        out_ref[:, :] = acc

        for o in range(1, N_DEV):
            dst = lax.rem(my_pos + o, N_DEV)
            send = pltpu.make_async_remote_copy(
                src_ref=x_ref.at[pl.ds(dst * m_per, m_per), :],
                dst_ref=gather_ref.at[dst],
                send_sem=send_sems.at[dst],
                recv_sem=recv_sems.at[dst],
                device_id=(dst,),
                device_id_type=pl.DeviceIdType.MESH,
            )
            send.wait_send()

        pl.semaphore_wait(barrier_sem, N_DEV - 1)

    return pl.pallas_call(
        body,
        out_shape=jax.ShapeDtypeStruct((m_per, n_dim), jnp.float32),
        in_specs=[
            pl.BlockSpec(memory_space=pltpu.VMEM),
            pl.BlockSpec(memory_space=pltpu.VMEM),
        ],
        out_specs=pl.BlockSpec(memory_space=pltpu.VMEM),
        scratch_shapes=[
            pltpu.VMEM((N_DEV, m_per, m_per), jnp.float32),
            pltpu.SemaphoreType.DMA((N_DEV,)),
            pltpu.SemaphoreType.DMA((N_DEV,)),
            pltpu.SemaphoreType.REGULAR((N_DEV,)),
        ],
        compiler_params=pltpu.CompilerParams(collective_id=0),
    )(x, w_mat)


# device time: 12405 ns/iter; 1.3316x vs baseline; 1.0888x over previous
import jax
import jax.numpy as jnp
from jax import lax
from jax.experimental import pallas as pl
from jax.experimental.pallas import tpu as pltpu

N_DEV = 8


def kernel(x, w_mat):
    k_dim, m_per = x.shape
    n_dim = w_mat.shape[1]

    def body(x_ref, w_ref, out_ref, xbf_ref, wbf_ref, gather_ref,
             send_sems, recv_sems, ready_sems):
        my_pos = lax.axis_index("i")
        barrier_sem = pltpu.get_barrier_semaphore()

        for j in range(N_DEV):
            @pl.when(my_pos != j)
            def _():
                pl.semaphore_signal(
                    barrier_sem, inc=1, device_id=(j,),
                    device_id_type=pl.DeviceIdType.MESH,
                )
                pl.semaphore_signal(
                    ready_sems.at[my_pos], inc=1, device_id=(j,),
                    device_id_type=pl.DeviceIdType.MESH,
                )

        xbf_ref[:, :] = x_ref[:, :].astype(jnp.bfloat16)

        for o in range(1, N_DEV):
            dst = lax.rem(my_pos + o, N_DEV)
            pl.semaphore_wait(ready_sems.at[dst], 1)
            rdma = pltpu.make_async_remote_copy(
                src_ref=xbf_ref.at[pl.ds(dst * m_per, m_per), :],
                dst_ref=gather_ref.at[my_pos],
                send_sem=send_sems.at[dst],
                recv_sem=recv_sems.at[my_pos],
                device_id=(dst,),
                device_id_type=pl.DeviceIdType.MESH,
            )
            rdma.start()

        wbf_ref[:, :] = w_ref[:, :].astype(jnp.bfloat16)
        gather_ref[my_pos] = xbf_ref[pl.ds(my_pos * m_per, m_per), :]
        acc = jnp.dot(
            gather_ref[my_pos],
            wbf_ref[pl.ds(my_pos * m_per, m_per), :],
            preferred_element_type=jnp.float32,
        )

        for o in range(1, N_DEV):
            src = lax.rem(my_pos - o + N_DEV, N_DEV)
            recv = pltpu.make_async_remote_copy(
                src_ref=xbf_ref.at[pl.ds(src * m_per, m_per), :],
                dst_ref=gather_ref.at[src],
                send_sem=send_sems.at[src],
                recv_sem=recv_sems.at[src],
                device_id=(src,),
                device_id_type=pl.DeviceIdType.MESH,
            )
            recv.wait_recv()
            acc = acc + jnp.dot(
                gather_ref[src],
                wbf_ref[pl.ds(src * m_per, m_per), :],
                preferred_element_type=jnp.float32,
            )
        out_ref[:, :] = acc

        for o in range(1, N_DEV):
            dst = lax.rem(my_pos + o, N_DEV)
            send = pltpu.make_async_remote_copy(
                src_ref=xbf_ref.at[pl.ds(dst * m_per, m_per), :],
                dst_ref=gather_ref.at[dst],
                send_sem=send_sems.at[dst],
                recv_sem=recv_sems.at[dst],
                device_id=(dst,),
                device_id_type=pl.DeviceIdType.MESH,
            )
            send.wait_send()

        pl.semaphore_wait(barrier_sem, N_DEV - 1)

    return pl.pallas_call(
        body,
        out_shape=jax.ShapeDtypeStruct((m_per, n_dim), jnp.float32),
        in_specs=[
            pl.BlockSpec(memory_space=pltpu.VMEM),
            pl.BlockSpec(memory_space=pltpu.VMEM),
        ],
        out_specs=pl.BlockSpec(memory_space=pltpu.VMEM),
        scratch_shapes=[
            pltpu.VMEM((k_dim, m_per), jnp.bfloat16),
            pltpu.VMEM((k_dim, n_dim), jnp.bfloat16),
            pltpu.VMEM((N_DEV, m_per, m_per), jnp.bfloat16),
            pltpu.SemaphoreType.DMA((N_DEV,)),
            pltpu.SemaphoreType.DMA((N_DEV,)),
            pltpu.SemaphoreType.REGULAR((N_DEV,)),
        ],
        compiler_params=pltpu.CompilerParams(collective_id=0),
    )(x, w_mat)
